# baseline (speedup 1.0000x reference)
import functools

import jax
import jax.numpy as jnp
from jax import lax
from jax.experimental import pallas as pl
from jax.experimental.pallas import tpu as pltpu
from jax.experimental.pallas import tpu_sc as plsc

NP_ = 576
PD_ = 768
B_ = 256

NC_ = 2
NS_ = 16
NW_ = NC_ * NS_
BPW_ = B_ // NW_
PC_ = 32
NPC_ = NP_ // PC_
NCHUNK_ = NPC_ * BPW_
NBUF_ = 4
NVREG_ = PC_ * PD_ // 16
VPR_ = PD_ // 16

_mesh = plsc.VectorSubcoreMesh(core_axis_name="c", subcore_axis_name="s")


@functools.partial(
    pl.kernel,
    out_type=jax.ShapeDtypeStruct((B_, NP_, PD_), jnp.float32),
    mesh=_mesh,
    scratch_types=(
        [pltpu.VMEM((PC_, PD_), jnp.float32)]
        + [pltpu.VMEM((PC_, PD_), jnp.float32) for _ in range(NBUF_)]
        + [pltpu.SemaphoreType.DMA for _ in range(2 * NBUF_)]
    ),
)
def _sc_add(x_hbm, t_hbm, out_hbm, t_v, *bufs_and_sems):
    bufs = bufs_and_sems[:NBUF_]
    in_sems = bufs_and_sems[NBUF_:2 * NBUF_]
    out_sems = bufs_and_sems[2 * NBUF_:]

    w = lax.axis_index("s") * NC_ + lax.axis_index("c")
    b0 = w * BPW_

    def x_slice(ref, u):
        pc = u // BPW_
        b = b0 + (u % BPW_)
        return ref.at[b, pl.ds(pc * PC_, PC_)]

    pltpu.async_copy(x_slice(x_hbm, 0), bufs[0], in_sems[0])
    pltpu.async_copy(x_slice(x_hbm, 1), bufs[1], in_sems[1])

    def step(i, carry):
        for ph in range(NBUF_):
            u = i * NBUF_ + ph
            bph = (ph + NBUF_ // 2) % NBUF_

            if ph == 0:
                def load_t():
                    pc = i // 2
                    pltpu.sync_copy(t_hbm.at[pl.ds(pc * PC_, PC_)], t_v)

                pl.when(i % 2 == 0)(load_t)

            def drain_buddy():
                pltpu.make_async_copy(
                    bufs[bph], x_slice(out_hbm, u - NBUF_ // 2), out_sems[bph]
                ).wait()

            def prefetch_buddy():
                pltpu.async_copy(
                    x_slice(x_hbm, u + NBUF_ // 2), bufs[bph], in_sems[bph]
                )

            if ph < NBUF_ // 2:
                pl.when(i >= 1)(drain_buddy)
                prefetch_buddy()
            else:
                drain_buddy()
                pl.when(i < NCHUNK_ // NBUF_ - 1)(prefetch_buddy)

            pltpu.make_async_copy(x_slice(x_hbm, u), bufs[ph], in_sems[ph]).wait()

            buf = bufs[ph]

            @plsc.parallel_loop(0, NVREG_, unroll=8)
            def add_body(j):
                r = j // VPR_
                col = (j % VPR_) * 16
                plsc.addupdate(buf.at[r, pl.ds(col, 16)], t_v[r, pl.ds(col, 16)])

            pltpu.async_copy(buf, x_slice(out_hbm, u), out_sems[ph])
        return carry

    lax.fori_loop(0, NCHUNK_ // NBUF_, step, 0)

    for ph in range(NBUF_ // 2, NBUF_):
        u = NCHUNK_ - NBUF_ + ph
        pltpu.make_async_copy(bufs[ph], x_slice(out_hbm, u), out_sems[ph]).wait()


def kernel(encoded_patches, pos_table):
    return _sc_add(encoded_patches, pos_table)


# speedup vs baseline: 1.3785x; 1.3785x over previous
import functools

import jax
import jax.numpy as jnp
from jax import lax
from jax.experimental import pallas as pl
from jax.experimental.pallas import tpu as pltpu
from jax.experimental.pallas import tpu_sc as plsc

NP_ = 576
PD_ = 768
B_ = 256

NC_ = 2
NS_ = 16
NW_ = NC_ * NS_
BPW_ = B_ // NW_
PC_ = 32
NPC_ = NP_ // PC_
NCHUNK_ = NPC_ * BPW_
NBUF_ = 4
NVREG_ = PC_ * PD_ // 16
VPR_ = PD_ // 16

_mesh = plsc.VectorSubcoreMesh(core_axis_name="c", subcore_axis_name="s")


@functools.partial(
    pl.kernel,
    out_type=jax.ShapeDtypeStruct((B_, NP_, PD_), jnp.float32),
    mesh=_mesh,
    scratch_types=(
        [pltpu.VMEM((PC_, PD_), jnp.float32)]
        + [pltpu.VMEM((PC_, PD_), jnp.float32) for _ in range(NBUF_)]
        + [pltpu.SemaphoreType.DMA for _ in range(2 * NBUF_)]
    ),
)
def _sc_add(x_hbm, t_hbm, out_hbm, t_v, *bufs_and_sems):
    bufs = bufs_and_sems[:NBUF_]
    in_sems = bufs_and_sems[NBUF_:2 * NBUF_]
    out_sems = bufs_and_sems[2 * NBUF_:]

    w = lax.axis_index("s") * NC_ + lax.axis_index("c")
    b0 = w * BPW_

    def x_slice(ref, u):
        pc = u // BPW_
        b = b0 + (u % BPW_)
        return ref.at[b, pl.ds(pc * PC_, PC_)]

    pltpu.async_copy(x_slice(x_hbm, 0), bufs[0], in_sems[0])
    pltpu.async_copy(x_slice(x_hbm, 1), bufs[1], in_sems[1])

    def step(i, carry):
        for ph in range(NBUF_):
            u = i * NBUF_ + ph
            bph = (ph + NBUF_ // 2) % NBUF_

            if ph == 0:
                def load_t():
                    pc = i // 2
                    pltpu.sync_copy(t_hbm.at[pl.ds(pc * PC_, PC_)], t_v)

                pl.when(i % 2 == 0)(load_t)

            def drain_buddy():
                pltpu.make_async_copy(
                    bufs[bph], x_slice(out_hbm, u - NBUF_ // 2), out_sems[bph]
                ).wait()

            def prefetch_buddy():
                pltpu.async_copy(
                    x_slice(x_hbm, u + NBUF_ // 2), bufs[bph], in_sems[bph]
                )

            if ph < NBUF_ // 2:
                pl.when(i >= 1)(drain_buddy)
                prefetch_buddy()
            else:
                drain_buddy()
                pl.when(i < NCHUNK_ // NBUF_ - 1)(prefetch_buddy)

            pltpu.make_async_copy(x_slice(x_hbm, u), bufs[ph], in_sems[ph]).wait()

            buf = bufs[ph]

            @plsc.parallel_loop(0, PC_, unroll=2)
            def add_body(r):
                for c in range(VPR_):
                    plsc.addupdate(
                        buf.at[r, pl.ds(c * 16, 16)], t_v[r, pl.ds(c * 16, 16)]
                    )

            pltpu.async_copy(buf, x_slice(out_hbm, u), out_sems[ph])
        return carry

    lax.fori_loop(0, NCHUNK_ // NBUF_, step, 0)

    for ph in range(NBUF_ // 2, NBUF_):
        u = NCHUNK_ - NBUF_ + ph
        pltpu.make_async_copy(bufs[ph], x_slice(out_hbm, u), out_sems[ph]).wait()


def kernel(encoded_patches, pos_table):
    return _sc_add(encoded_patches, pos_table)


# speedup vs baseline: 1.9097x; 1.3853x over previous
import jax
import jax.numpy as jnp
from jax.experimental import pallas as pl

NP_ = 576
PD_ = 768
B_ = 256
BLOCK_B = 8


def _add_kernel(x_ref, t_ref, o_ref):
    o_ref[...] = x_ref[...] + t_ref[...]


def kernel(encoded_patches, pos_table):
    grid = (B_ // BLOCK_B,)
    return pl.pallas_call(
        _add_kernel,
        grid=grid,
        in_specs=[
            pl.BlockSpec((BLOCK_B, NP_, PD_), lambda i: (i, 0, 0)),
            pl.BlockSpec((NP_, PD_), lambda i: (0, 0)),
        ],
        out_specs=pl.BlockSpec((BLOCK_B, NP_, PD_), lambda i: (i, 0, 0)),
        out_shape=jax.ShapeDtypeStruct((B_, NP_, PD_), jnp.float32),
    )(encoded_patches, pos_table)
